# baseline (speedup 1.0000x reference)
import functools

import jax
import jax.numpy as jnp
from jax import lax
from jax.experimental import pallas as pl
from jax.experimental.pallas import tpu as pltpu
from jax.experimental.pallas import tpu_sc as plsc

_NC = 2
_NS = 16
_NW = _NC * _NS

_B = 80
_CB = 5
_CBS = 25
_RPW = 640
_ZC = 80
_NPAD = _NS * _RPW


def _scatter_kernel(d, n_chunks):
    def body(x_hbm, src_hbm, dst_hbm, rowidx_hbm, zrow_hbm, s_out,
             src_v, dst_v, ridx_v, rows0_v, rows1_v,
             shared_s, sem_g0, sem_g1, sem_s0, sem_s1):
        cid = lax.axis_index("c")
        sid = lax.axis_index("s")
        wid = sid * _NC + cid
        r0 = pl.multiple_of(sid * _RPW, 8)
        rows = (rows0_v, rows1_v)
        sem_g = (sem_g0, sem_g1)
        sem_s = (sem_s0, sem_s1)

        pltpu.sync_copy(rowidx_hbm.at[sid], ridx_v)
        pltpu.sync_copy(zrow_hbm, rows0_v)
        for t in range(_RPW // _ZC):
            pltpu.sync_copy(rows0_v, shared_s.at[ridx_v.at[t]])
        plsc.subcore_barrier()

        def block(b, carry):
            pltpu.sync_copy(src_hbm.at[wid, b], src_v)
            pltpu.sync_copy(dst_hbm.at[wid, b], dst_v)
            gd = [None, None]
            sd = [None, None]
            gd[0] = pltpu.async_copy(x_hbm.at[src_v.at[0]], rows[0],
                                     sem_g[0])
            for j in range(_CBS):
                p = j % 2
                gd[p].wait()
                if j + 1 < _CBS:
                    q = (j + 1) % 2
                    if sd[q] is not None:
                        sd[q].wait()
                    gd[q] = pltpu.async_copy(x_hbm.at[src_v.at[j + 1]],
                                             rows[q], sem_g[q])
                sd[p] = pltpu.async_copy(rows[p],
                                         shared_s.at[dst_v.at[j]],
                                         sem_s[p], add=True)
            for p in range(2):
                if sd[p] is not None:
                    sd[p].wait()
            return carry

        lax.fori_loop(0, n_chunks // _CBS, block, 0)
        plsc.subcore_barrier()

        for t in range(_RPW // _ZC):
            rs = pl.multiple_of(r0 + t * _ZC, 8)
            pltpu.async_copy(shared_s.at[ridx_v.at[t]], rows0_v,
                             sem_g0).wait()
            pltpu.sync_copy(rows0_v, s_out.at[cid, pl.ds(rs, _ZC)])

    return pl.kernel(
        body,
        out_type=jax.ShapeDtypeStruct((_NC, _NPAD, d), jnp.float32),
        mesh=plsc.VectorSubcoreMesh(core_axis_name="c", subcore_axis_name="s"),
        scratch_types=(
            pltpu.VMEM((_CBS, _B), jnp.int32),
            pltpu.VMEM((_CBS, _B), jnp.int32),
            pltpu.VMEM((_RPW // _ZC, _ZC), jnp.int32),
            pltpu.VMEM((_B, d), jnp.float32),
            pltpu.VMEM((_B, d), jnp.float32),
            pltpu.VMEM_SHARED((_NPAD, d), jnp.float32),
            pltpu.SemaphoreType.DMA,
            pltpu.SemaphoreType.DMA,
            pltpu.SemaphoreType.DMA,
            pltpu.SemaphoreType.DMA,
        ),
    )


_EB = 2000


def _hist_body(dst_ref, deg_ref):
    i = pl.program_id(0)

    @pl.when(i == 0)
    def _():
        deg_ref[...] = jnp.zeros_like(deg_ref)

    dsts = dst_ref[0]
    hi = dsts // 128
    lo = dsts % 128
    ohi = (lax.broadcasted_iota(jnp.int32, (_EB, _NPAD // 128), 1)
           == hi).astype(jnp.bfloat16)
    olo = (lax.broadcasted_iota(jnp.int32, (_EB, 128), 1)
           == lo).astype(jnp.bfloat16)
    deg_ref[...] += lax.dot_general(
        ohi, olo, (((0,), (0,)), ((), ())),
        preferred_element_type=jnp.float32)


def _leaky(t):
    return jnp.where(t >= 0, t, 0.01 * t)


def _dense_body(x_ref, s_ref, deg_ref, wself_ref, wnbr_ref, bg_ref, w1_ref,
                b1_ref, w2_ref, b2_ref, wk_ref, bk_ref, ww_ref, bw_ref,
                z_ref, keys_ref, wout_ref):
    x = x_ref[...]
    s = s_ref[0] + s_ref[1]
    deg = deg_ref[...]
    agg = s / jnp.maximum(deg, 1.0)
    dot = functools.partial(jnp.dot, preferred_element_type=jnp.float32)
    h = dot(x, wself_ref[...]) + dot(agg, wnbr_ref[...]) + bg_ref[...]
    z1 = dot(_leaky(h), w1_ref[...]) + b1_ref[...]
    z_ref[...] = dot(_leaky(z1), w2_ref[...]) + b2_ref[...]
    keys_ref[...] = dot(x, wk_ref[...]) + bk_ref[...]
    wout_ref[...] = dot(x, ww_ref[...]) + bw_ref[...]


def kernel(x, edge, W_self, W_nbr, b_gnn, W1, b1, W2, b2, Wk, bk, Ww, bw):
    n, d = x.shape
    e = edge.shape[1]
    epw = e // _NW
    assert e % _NW == 0 and epw % (_B * _CB) == 0
    assert n <= _NPAD
    n_chunks = epw // _B

    src_rs = edge[0].reshape(_NW, n_chunks // _CBS, _CBS, _B)
    dst_rs = edge[1].reshape(_NW, n_chunks // _CBS, _CBS, _B)
    dst_b = edge[1].reshape(e // _EB, _EB, 1)
    rowidx = jnp.arange(_NPAD, dtype=jnp.int32).reshape(
        _NS, _RPW // _ZC, _ZC)
    zrow = jnp.zeros((_ZC, d), jnp.float32)

    s_part = _scatter_kernel(d, n_chunks)(x, src_rs, dst_rs, rowidx, zrow)

    deg_mat = pl.pallas_call(
        _hist_body,
        grid=(e // _EB,),
        in_specs=[pl.BlockSpec((1, _EB, 1), lambda i: (i, 0, 0))],
        out_specs=pl.BlockSpec((_NPAD // 128, 128), lambda i: (0, 0)),
        out_shape=jax.ShapeDtypeStruct((_NPAD // 128, 128), jnp.float32),
    )(dst_b)
    deg_vec = deg_mat.reshape(_NPAD, 1)

    d_z = W2.shape[1]
    blk = 1000
    grid = (n // blk,)
    full = lambda a: pl.BlockSpec(a.shape, lambda i: (0,) * a.ndim)
    z, keys, wout = pl.pallas_call(
        _dense_body,
        grid=grid,
        in_specs=[
            pl.BlockSpec((blk, d), lambda i: (i, 0)),
            pl.BlockSpec((_NC, blk, d), lambda i: (0, i, 0)),
            pl.BlockSpec((blk, 1), lambda i: (i, 0)),
            full(W_self), full(W_nbr),
            pl.BlockSpec((1, d), lambda i: (0, 0)),
            full(W1),
            pl.BlockSpec((1, d), lambda i: (0, 0)),
            full(W2),
            pl.BlockSpec((1, d_z), lambda i: (0, 0)),
            full(Wk),
            pl.BlockSpec((1, d_z), lambda i: (0, 0)),
            full(Ww),
            pl.BlockSpec((1, 1), lambda i: (0, 0)),
        ],
        out_specs=[
            pl.BlockSpec((blk, d_z), lambda i: (i, 0)),
            pl.BlockSpec((blk, d_z), lambda i: (i, 0)),
            pl.BlockSpec((blk, 1), lambda i: (i, 0)),
        ],
        out_shape=[
            jax.ShapeDtypeStruct((n, d_z), jnp.float32),
            jax.ShapeDtypeStruct((n, d_z), jnp.float32),
            jax.ShapeDtypeStruct((n, 1), jnp.float32),
        ],
    )(x, s_part, deg_vec, W_self, W_nbr, b_gnn.reshape(1, d), W1,
      b1.reshape(1, d), W2, b2.reshape(1, d_z), Wk, bk.reshape(1, d_z), Ww,
      bw.reshape(1, 1))
    return (z, keys, wout)


# speedup vs baseline: 1.1720x; 1.1720x over previous
import functools

import jax
import jax.numpy as jnp
from jax import lax
from jax.experimental import pallas as pl
from jax.experimental.pallas import tpu as pltpu
from jax.experimental.pallas import tpu_sc as plsc

_NC = 2
_NS = 16
_NW = _NC * _NS

_B = 80
_CB = 5
_CBS = 25
_RPW = 640
_ZC = 80
_NPAD = _NS * _RPW


def _scatter_kernel(d, n_chunks):
    def body(x_hbm, src_hbm, dst_hbm, rowidx_hbm, zrow_hbm, s_out,
             src_v, dst_v, ridx_v, rows0_v, rows1_v,
             shared_s, sem_g0, sem_g1, sem_s0, sem_s1):
        cid = lax.axis_index("c")
        sid = lax.axis_index("s")
        wid = sid * _NC + cid
        r0 = pl.multiple_of(sid * _RPW, 8)
        rows = (rows0_v, rows1_v)
        sem_g = (sem_g0, sem_g1)
        sem_s = (sem_s0, sem_s1)

        pltpu.sync_copy(rowidx_hbm.at[sid], ridx_v)
        pltpu.sync_copy(zrow_hbm, rows0_v)
        for t in range(_RPW // _ZC):
            pltpu.sync_copy(rows0_v, shared_s.at[ridx_v.at[t]])
        plsc.subcore_barrier()

        def block(b, carry):
            pltpu.sync_copy(src_hbm.at[wid, b], src_v)
            pltpu.sync_copy(dst_hbm.at[wid, b], dst_v)
            gd = [None, None]
            sd = [None, None]
            gd[0] = pltpu.async_copy(x_hbm.at[src_v.at[0]], rows[0],
                                     sem_g[0])
            for j in range(_CBS):
                p = j % 2
                gd[p].wait()
                if j + 1 < _CBS:
                    q = (j + 1) % 2
                    if sd[q] is not None:
                        sd[q].wait()
                    gd[q] = pltpu.async_copy(x_hbm.at[src_v.at[j + 1]],
                                             rows[q], sem_g[q])
                sd[p] = pltpu.async_copy(rows[p],
                                         shared_s.at[dst_v.at[j]],
                                         sem_s[p], add=True)
            for p in range(2):
                if sd[p] is not None:
                    sd[p].wait()
            return carry

        lax.fori_loop(0, n_chunks // _CBS, block, 0)
        plsc.subcore_barrier()

        for t in range(_RPW // _ZC):
            rs = pl.multiple_of(r0 + t * _ZC, 8)
            pltpu.async_copy(shared_s.at[ridx_v.at[t]], rows0_v,
                             sem_g0).wait()
            pltpu.sync_copy(rows0_v, s_out.at[cid, pl.ds(rs, _ZC)])

    return pl.kernel(
        body,
        out_type=jax.ShapeDtypeStruct((_NC, _NPAD, d), jnp.float32),
        mesh=plsc.VectorSubcoreMesh(core_axis_name="c", subcore_axis_name="s"),
        scratch_types=(
            pltpu.VMEM((_CBS, _B), jnp.int32),
            pltpu.VMEM((_CBS, _B), jnp.int32),
            pltpu.VMEM((_RPW // _ZC, _ZC), jnp.int32),
            pltpu.VMEM((_B, d), jnp.float32),
            pltpu.VMEM((_B, d), jnp.float32),
            pltpu.VMEM_SHARED((_NPAD, d), jnp.float32),
            pltpu.SemaphoreType.DMA,
            pltpu.SemaphoreType.DMA,
            pltpu.SemaphoreType.DMA,
            pltpu.SemaphoreType.DMA,
        ),
    )


_EB = 8000


def _hist_body(dst_ref, deg_ref):
    i = pl.program_id(0)

    @pl.when(i == 0)
    def _():
        deg_ref[...] = jnp.zeros_like(deg_ref)

    dsts = dst_ref[0]
    hi = lax.shift_right_logical(dsts, 7)
    lo = jnp.bitwise_and(dsts, 127)
    ohi = (lax.broadcasted_iota(jnp.int32, (_EB, _NPAD // 128), 1)
           == hi).astype(jnp.bfloat16)
    olo = (lax.broadcasted_iota(jnp.int32, (_EB, 128), 1)
           == lo).astype(jnp.bfloat16)
    deg_ref[...] += lax.dot_general(
        ohi, olo, (((0,), (0,)), ((), ())),
        preferred_element_type=jnp.float32)


def _leaky(t):
    return jnp.where(t >= 0, t, 0.01 * t)


def _dense_body(x_ref, s_ref, deg_ref, wself_ref, wnbr_ref, bg_ref, w1_ref,
                b1_ref, w2_ref, b2_ref, wk_ref, bk_ref, ww_ref, bw_ref,
                z_ref, keys_ref, wout_ref):
    x = x_ref[...]
    s = s_ref[0] + s_ref[1]
    deg = deg_ref[...]
    agg = s / jnp.maximum(deg, 1.0)
    dot = functools.partial(jnp.dot, preferred_element_type=jnp.float32)
    h = dot(x, wself_ref[...]) + dot(agg, wnbr_ref[...]) + bg_ref[...]
    z1 = dot(_leaky(h), w1_ref[...]) + b1_ref[...]
    z_ref[...] = dot(_leaky(z1), w2_ref[...]) + b2_ref[...]
    keys_ref[...] = dot(x, wk_ref[...]) + bk_ref[...]
    wout_ref[...] = dot(x, ww_ref[...]) + bw_ref[...]


def kernel(x, edge, W_self, W_nbr, b_gnn, W1, b1, W2, b2, Wk, bk, Ww, bw):
    n, d = x.shape
    e = edge.shape[1]
    epw = e // _NW
    assert e % _NW == 0 and epw % (_B * _CB) == 0
    assert n <= _NPAD
    n_chunks = epw // _B

    src_rs = edge[0].reshape(_NW, n_chunks // _CBS, _CBS, _B)
    dst_rs = edge[1].reshape(_NW, n_chunks // _CBS, _CBS, _B)
    dst_b = edge[1].reshape(e // _EB, _EB, 1)
    rowidx = jnp.arange(_NPAD, dtype=jnp.int32).reshape(
        _NS, _RPW // _ZC, _ZC)
    zrow = jnp.zeros((_ZC, d), jnp.float32)

    s_part = _scatter_kernel(d, n_chunks)(x, src_rs, dst_rs, rowidx, zrow)

    deg_mat = pl.pallas_call(
        _hist_body,
        grid=(e // _EB,),
        in_specs=[pl.BlockSpec((1, _EB, 1), lambda i: (i, 0, 0))],
        out_specs=pl.BlockSpec((_NPAD // 128, 128), lambda i: (0, 0)),
        out_shape=jax.ShapeDtypeStruct((_NPAD // 128, 128), jnp.float32),
    )(dst_b)
    deg_vec = deg_mat.reshape(_NPAD, 1)

    d_z = W2.shape[1]
    blk = 1000
    grid = (n // blk,)
    full = lambda a: pl.BlockSpec(a.shape, lambda i: (0,) * a.ndim)
    z, keys, wout = pl.pallas_call(
        _dense_body,
        grid=grid,
        in_specs=[
            pl.BlockSpec((blk, d), lambda i: (i, 0)),
            pl.BlockSpec((_NC, blk, d), lambda i: (0, i, 0)),
            pl.BlockSpec((blk, 1), lambda i: (i, 0)),
            full(W_self), full(W_nbr),
            pl.BlockSpec((1, d), lambda i: (0, 0)),
            full(W1),
            pl.BlockSpec((1, d), lambda i: (0, 0)),
            full(W2),
            pl.BlockSpec((1, d_z), lambda i: (0, 0)),
            full(Wk),
            pl.BlockSpec((1, d_z), lambda i: (0, 0)),
            full(Ww),
            pl.BlockSpec((1, 1), lambda i: (0, 0)),
        ],
        out_specs=[
            pl.BlockSpec((blk, d_z), lambda i: (i, 0)),
            pl.BlockSpec((blk, d_z), lambda i: (i, 0)),
            pl.BlockSpec((blk, 1), lambda i: (i, 0)),
        ],
        out_shape=[
            jax.ShapeDtypeStruct((n, d_z), jnp.float32),
            jax.ShapeDtypeStruct((n, d_z), jnp.float32),
            jax.ShapeDtypeStruct((n, 1), jnp.float32),
        ],
    )(x, s_part, deg_vec, W_self, W_nbr, b_gnn.reshape(1, d), W1,
      b1.reshape(1, d), W2, b2.reshape(1, d_z), Wk, bk.reshape(1, d_z), Ww,
      bw.reshape(1, 1))
    return (z, keys, wout)


# speedup vs baseline: 1.6301x; 1.3908x over previous
import functools

import jax
import jax.numpy as jnp
from jax import lax
from jax.experimental import pallas as pl
from jax.experimental.pallas import tpu as pltpu
from jax.experimental.pallas import tpu_sc as plsc

_NC = 2
_NS = 16
_NW = _NC * _NS

_B = 80
_CB = 5
_CBS = 25
_RPW = 640
_ZC = 80
_NPAD = _NS * _RPW


def _scatter_kernel(d, n_chunks):
    def body(x_hbm, src_hbm, dst_hbm, rowidx_hbm, zrow_hbm, s_out,
             src_v, dst_v, ridx_v, rows0_v, rows1_v,
             shared_s, sem_g0, sem_g1, sem_s0, sem_s1):
        cid = lax.axis_index("c")
        sid = lax.axis_index("s")
        wid = sid * _NC + cid
        r0 = pl.multiple_of(sid * _RPW, 8)
        rows = (rows0_v, rows1_v)
        sem_g = (sem_g0, sem_g1)
        sem_s = (sem_s0, sem_s1)

        pltpu.sync_copy(rowidx_hbm.at[sid], ridx_v)
        pltpu.sync_copy(zrow_hbm, rows0_v)
        for t in range(_RPW // _ZC):
            pltpu.sync_copy(rows0_v, shared_s.at[ridx_v.at[t]])
        plsc.subcore_barrier()

        def block(b, carry):
            pltpu.sync_copy(src_hbm.at[wid, b], src_v)
            pltpu.sync_copy(dst_hbm.at[wid, b], dst_v)
            gd = [None, None]
            sd = [None, None]
            gd[0] = pltpu.async_copy(x_hbm.at[src_v.at[0]], rows[0],
                                     sem_g[0])
            for j in range(_CBS):
                p = j % 2
                gd[p].wait()
                if j + 1 < _CBS:
                    q = (j + 1) % 2
                    if sd[q] is not None:
                        sd[q].wait()
                    gd[q] = pltpu.async_copy(x_hbm.at[src_v.at[j + 1]],
                                             rows[q], sem_g[q])
                sd[p] = pltpu.async_copy(rows[p],
                                         shared_s.at[dst_v.at[j]],
                                         sem_s[p], add=True)
            for p in range(2):
                if sd[p] is not None:
                    sd[p].wait()
            return carry

        lax.fori_loop(0, n_chunks // _CBS, block, 0)
        plsc.subcore_barrier()

        for t in range(_RPW // _ZC):
            rs = pl.multiple_of(r0 + t * _ZC, 8)
            pltpu.async_copy(shared_s.at[ridx_v.at[t]], rows0_v,
                             sem_g0).wait()
            pltpu.sync_copy(rows0_v, s_out.at[cid, pl.ds(rs, _ZC)])

    return pl.kernel(
        body,
        out_type=jax.ShapeDtypeStruct((_NC, _NPAD, d), jnp.float32),
        mesh=plsc.VectorSubcoreMesh(core_axis_name="c", subcore_axis_name="s"),
        scratch_types=(
            pltpu.VMEM((_CBS, _B), jnp.int32),
            pltpu.VMEM((_CBS, _B), jnp.int32),
            pltpu.VMEM((_RPW // _ZC, _ZC), jnp.int32),
            pltpu.VMEM((_B, d), jnp.float32),
            pltpu.VMEM((_B, d), jnp.float32),
            pltpu.VMEM_SHARED((_NPAD, d), jnp.float32),
            pltpu.SemaphoreType.DMA,
            pltpu.SemaphoreType.DMA,
            pltpu.SemaphoreType.DMA,
            pltpu.SemaphoreType.DMA,
        ),
    )


def _deg_kernel(d, n_chunks):
    def body(dst_hbm, rowidx_hbm, zrow_hbm, ones_hbm, deg_out,
             dst_v, ridx_v, rows_v, ones_v, shared_deg, sem):
        cid = lax.axis_index("c")
        sid = lax.axis_index("s")
        wid = sid * _NC + cid
        r0 = pl.multiple_of(sid * _RPW, 8)

        pltpu.sync_copy(rowidx_hbm.at[sid], ridx_v)
        pltpu.sync_copy(zrow_hbm, rows_v)
        for t in range(_RPW // _ZC):
            pltpu.sync_copy(rows_v, shared_deg.at[ridx_v.at[t]])
        pltpu.sync_copy(ones_hbm, ones_v)
        plsc.subcore_barrier()

        def block(b, carry):
            pltpu.sync_copy(dst_hbm.at[wid, b], dst_v)
            for g in range(_CBS // 5):
                descs = [
                    pltpu.async_copy(ones_v,
                                     shared_deg.at[dst_v.at[5 * g + j]],
                                     sem, add=True)
                    for j in range(5)
                ]
                for dsc in descs:
                    dsc.wait()
            return carry

        lax.fori_loop(0, n_chunks // _CBS, block, 0)
        plsc.subcore_barrier()

        for t in range(_RPW // _ZC):
            rs = pl.multiple_of(r0 + t * _ZC, 8)
            pltpu.async_copy(shared_deg.at[ridx_v.at[t]], rows_v,
                             sem).wait()
            pltpu.sync_copy(rows_v, deg_out.at[cid, pl.ds(rs, _ZC)])

    return pl.kernel(
        body,
        out_type=jax.ShapeDtypeStruct((_NC, _NPAD, d), jnp.float32),
        mesh=plsc.VectorSubcoreMesh(core_axis_name="c", subcore_axis_name="s"),
        scratch_types=(
            pltpu.VMEM((_CBS, _B), jnp.int32),
            pltpu.VMEM((_RPW // _ZC, _ZC), jnp.int32),
            pltpu.VMEM((_B, d), jnp.float32),
            pltpu.VMEM((_B, d), jnp.float32),
            pltpu.VMEM_SHARED((_NPAD, d), jnp.float32),
            pltpu.SemaphoreType.DMA,
        ),
    )


def _leaky(t):
    return jnp.where(t >= 0, t, 0.01 * t)


def _dense_body(x_ref, s_ref, deg_ref, wself_ref, wnbr_ref, bg_ref, w1_ref,
                b1_ref, w2_ref, b2_ref, wk_ref, bk_ref, ww_ref, bw_ref,
                z_ref, keys_ref, wout_ref):
    x = x_ref[...]
    s = s_ref[0] + s_ref[1]
    deg = deg_ref[0, :, 0:1] + deg_ref[1, :, 0:1]
    agg = s / jnp.maximum(deg, 1.0)
    dot = functools.partial(jnp.dot, preferred_element_type=jnp.float32)
    h = dot(x, wself_ref[...]) + dot(agg, wnbr_ref[...]) + bg_ref[...]
    z1 = dot(_leaky(h), w1_ref[...]) + b1_ref[...]
    z_ref[...] = dot(_leaky(z1), w2_ref[...]) + b2_ref[...]
    keys_ref[...] = dot(x, wk_ref[...]) + bk_ref[...]
    wout_ref[...] = dot(x, ww_ref[...]) + bw_ref[...]


def kernel(x, edge, W_self, W_nbr, b_gnn, W1, b1, W2, b2, Wk, bk, Ww, bw):
    n, d = x.shape
    e = edge.shape[1]
    epw = e // _NW
    assert e % _NW == 0 and epw % (_B * _CB) == 0
    assert n <= _NPAD
    n_chunks = epw // _B

    src_rs = edge[0].reshape(_NW, n_chunks // _CBS, _CBS, _B)
    dst_rs = edge[1].reshape(_NW, n_chunks // _CBS, _CBS, _B)
    rowidx = jnp.arange(_NPAD, dtype=jnp.int32).reshape(
        _NS, _RPW // _ZC, _ZC)
    zrow = jnp.zeros((_ZC, d), jnp.float32)
    ones = jnp.ones((_B, d), jnp.float32)

    s_part = _scatter_kernel(d, n_chunks)(x, src_rs, dst_rs, rowidx, zrow)
    deg_part = _deg_kernel(d, n_chunks)(dst_rs, rowidx, zrow, ones)

    d_z = W2.shape[1]
    blk = 1000
    grid = (n // blk,)
    full = lambda a: pl.BlockSpec(a.shape, lambda i: (0,) * a.ndim)
    z, keys, wout = pl.pallas_call(
        _dense_body,
        grid=grid,
        in_specs=[
            pl.BlockSpec((blk, d), lambda i: (i, 0)),
            pl.BlockSpec((_NC, blk, d), lambda i: (0, i, 0)),
            pl.BlockSpec((_NC, blk, d), lambda i: (0, i, 0)),
            full(W_self), full(W_nbr),
            pl.BlockSpec((1, d), lambda i: (0, 0)),
            full(W1),
            pl.BlockSpec((1, d), lambda i: (0, 0)),
            full(W2),
            pl.BlockSpec((1, d_z), lambda i: (0, 0)),
            full(Wk),
            pl.BlockSpec((1, d_z), lambda i: (0, 0)),
            full(Ww),
            pl.BlockSpec((1, 1), lambda i: (0, 0)),
        ],
        out_specs=[
            pl.BlockSpec((blk, d_z), lambda i: (i, 0)),
            pl.BlockSpec((blk, d_z), lambda i: (i, 0)),
            pl.BlockSpec((blk, 1), lambda i: (i, 0)),
        ],
        out_shape=[
            jax.ShapeDtypeStruct((n, d_z), jnp.float32),
            jax.ShapeDtypeStruct((n, d_z), jnp.float32),
            jax.ShapeDtypeStruct((n, 1), jnp.float32),
        ],
    )(x, s_part, deg_part, W_self, W_nbr, b_gnn.reshape(1, d), W1,
      b1.reshape(1, d), W2, b2.reshape(1, d_z), Wk, bk.reshape(1, d_z), Ww,
      bw.reshape(1, 1))
    return (z, keys, wout)


# speedup vs baseline: 1.6310x; 1.0006x over previous
import functools

import jax
import jax.numpy as jnp
from jax import lax
from jax.experimental import pallas as pl
from jax.experimental.pallas import tpu as pltpu
from jax.experimental.pallas import tpu_sc as plsc

_NC = 2
_NS = 16
_NW = _NC * _NS

_B = 80
_CBS = 25
_RPW = 640
_ZC = 80
_NPAD = _NS * _RPW


def _scatter_kernel(d, n_chunks):
    def body(x_hbm, src_hbm, dst_hbm, rowidx_hbm, zrow_hbm, s_out,
             src_v, dst_v, ridx_v, rows0_v, rows1_v,
             shared_s, sem_g0, sem_g1, sem_s0, sem_s1):
        cid = lax.axis_index("c")
        sid = lax.axis_index("s")
        wid = sid * _NC + cid
        r0 = pl.multiple_of(sid * _RPW, 8)
        rows = (rows0_v, rows1_v)
        sem_g = (sem_g0, sem_g1)
        sem_s = (sem_s0, sem_s1)

        pltpu.sync_copy(rowidx_hbm.at[sid], ridx_v)
        pltpu.sync_copy(zrow_hbm, rows0_v)
        for t in range(_RPW // _ZC):
            pltpu.sync_copy(rows0_v, shared_s.at[ridx_v.at[t]])
        plsc.subcore_barrier()

        def block(b, carry):
            pltpu.sync_copy(src_hbm.at[wid, b], src_v)
            pltpu.sync_copy(dst_hbm.at[wid, b], dst_v)
            gd = [None, None]
            sd = [None, None]
            gd[0] = pltpu.async_copy(x_hbm.at[src_v.at[0]], rows[0],
                                     sem_g[0])
            for j in range(_CBS):
                p = j % 2
                gd[p].wait()
                if j + 1 < _CBS:
                    q = (j + 1) % 2
                    if sd[q] is not None:
                        sd[q].wait()
                    gd[q] = pltpu.async_copy(x_hbm.at[src_v.at[j + 1]],
                                             rows[q], sem_g[q])
                sd[p] = pltpu.async_copy(rows[p],
                                         shared_s.at[dst_v.at[j]],
                                         sem_s[p], add=True)
            for p in range(2):
                if sd[p] is not None:
                    sd[p].wait()
            return carry

        lax.fori_loop(0, n_chunks // _CBS, block, 0)
        plsc.subcore_barrier()

        for t in range(_RPW // _ZC):
            rs = pl.multiple_of(r0 + t * _ZC, 8)
            pltpu.async_copy(shared_s.at[ridx_v.at[t]], rows0_v,
                             sem_g0).wait()
            pltpu.sync_copy(rows0_v, s_out.at[cid, pl.ds(rs, _ZC)])

    return pl.kernel(
        body,
        out_type=jax.ShapeDtypeStruct((_NC, _NPAD, d), jnp.float32),
        mesh=plsc.VectorSubcoreMesh(core_axis_name="c", subcore_axis_name="s"),
        scratch_types=(
            pltpu.VMEM((_CBS, _B), jnp.int32),
            pltpu.VMEM((_CBS, _B), jnp.int32),
            pltpu.VMEM((_RPW // _ZC, _ZC), jnp.int32),
            pltpu.VMEM((_B, d), jnp.float32),
            pltpu.VMEM((_B, d), jnp.float32),
            pltpu.VMEM_SHARED((_NPAD, d), jnp.float32),
            pltpu.SemaphoreType.DMA,
            pltpu.SemaphoreType.DMA,
            pltpu.SemaphoreType.DMA,
            pltpu.SemaphoreType.DMA,
        ),
    )


def _deg_kernel(d, n_chunks):
    def body(dst_hbm, rowidx_hbm, zrow_hbm, ones_hbm, deg_out,
             dst_v, ridx_v, rows_v, ones_v, shared_deg, sem):
        cid = lax.axis_index("c")
        sid = lax.axis_index("s")
        wid = sid * _NC + cid
        r0 = pl.multiple_of(sid * _RPW, 8)

        pltpu.sync_copy(rowidx_hbm.at[sid], ridx_v)
        pltpu.sync_copy(zrow_hbm, rows_v)
        for t in range(_RPW // _ZC):
            pltpu.sync_copy(rows_v, shared_deg.at[ridx_v.at[t]])
        pltpu.sync_copy(ones_hbm, ones_v)
        plsc.subcore_barrier()

        def block(b, carry):
            pltpu.sync_copy(dst_hbm.at[wid, b], dst_v)
            for g in range(_CBS // 5):
                descs = [
                    pltpu.async_copy(ones_v,
                                     shared_deg.at[dst_v.at[5 * g + j]],
                                     sem, add=True)
                    for j in range(5)
                ]
                for dsc in descs:
                    dsc.wait()
            return carry

        lax.fori_loop(0, n_chunks // _CBS, block, 0)
        plsc.subcore_barrier()

        for t in range(_RPW // _ZC):
            rs = pl.multiple_of(r0 + t * _ZC, 8)
            pltpu.async_copy(shared_deg.at[ridx_v.at[t]], rows_v,
                             sem).wait()
            pltpu.sync_copy(rows_v, deg_out.at[cid, pl.ds(rs, _ZC)])

    return pl.kernel(
        body,
        out_type=jax.ShapeDtypeStruct((_NC, _NPAD, d), jnp.float32),
        mesh=plsc.VectorSubcoreMesh(core_axis_name="c", subcore_axis_name="s"),
        scratch_types=(
            pltpu.VMEM((_CBS, _B), jnp.int32),
            pltpu.VMEM((_RPW // _ZC, _ZC), jnp.int32),
            pltpu.VMEM((_B, d), jnp.float32),
            pltpu.VMEM((_B, d), jnp.float32),
            pltpu.VMEM_SHARED((_NPAD, d), jnp.float32),
            pltpu.SemaphoreType.DMA,
        ),
    )


def _leaky(t):
    return jnp.where(t >= 0, t, 0.01 * t)


def _dense_body(x_ref, s_ref, deg_ref, wself_ref, wnbr_ref, bg_ref, w1_ref,
                b1_ref, w2_ref, b2_ref, wk_ref, bk_ref, ww_ref, bw_ref,
                z_ref, keys_ref, wout_ref):
    x = x_ref[...]
    s = s_ref[0] + s_ref[1]
    deg = deg_ref[0, :, 0:1] + deg_ref[1, :, 0:1]
    agg = s / jnp.maximum(deg, 1.0)
    dot = functools.partial(jnp.dot, preferred_element_type=jnp.float32)
    h = dot(x, wself_ref[...]) + dot(agg, wnbr_ref[...]) + bg_ref[...]
    z1 = dot(_leaky(h), w1_ref[...]) + b1_ref[...]
    z_ref[...] = dot(_leaky(z1), w2_ref[...]) + b2_ref[...]
    keys_ref[...] = dot(x, wk_ref[...]) + bk_ref[...]
    wout_ref[...] = dot(x, ww_ref[...]) + bw_ref[...]


def kernel(x, edge, W_self, W_nbr, b_gnn, W1, b1, W2, b2, Wk, bk, Ww, bw):
    n, d = x.shape
    e = edge.shape[1]
    epw = e // _NW
    assert e % _NW == 0 and epw % (_B * _CBS) == 0
    assert n <= _NPAD
    n_chunks = epw // _B

    src_rs = edge[0].reshape(_NW, n_chunks // _CBS, _CBS, _B)
    dst_rs = edge[1].reshape(_NW, n_chunks // _CBS, _CBS, _B)
    rowidx = jnp.arange(_NPAD, dtype=jnp.int32).reshape(
        _NS, _RPW // _ZC, _ZC)
    zrow = jnp.zeros((_ZC, d), jnp.float32)
    ones = jnp.ones((_B, d), jnp.float32)

    s_part = _scatter_kernel(d, n_chunks)(x, src_rs, dst_rs, rowidx, zrow)
    deg_part = _deg_kernel(d, n_chunks)(dst_rs, rowidx, zrow, ones)

    d_z = W2.shape[1]
    blk = 1000
    grid = (n // blk,)
    full = lambda a: pl.BlockSpec(a.shape, lambda i: (0,) * a.ndim)
    z, keys, wout = pl.pallas_call(
        _dense_body,
        grid=grid,
        in_specs=[
            pl.BlockSpec((blk, d), lambda i: (i, 0)),
            pl.BlockSpec((_NC, blk, d), lambda i: (0, i, 0)),
            pl.BlockSpec((_NC, blk, d), lambda i: (0, i, 0)),
            full(W_self), full(W_nbr),
            pl.BlockSpec((1, d), lambda i: (0, 0)),
            full(W1),
            pl.BlockSpec((1, d), lambda i: (0, 0)),
            full(W2),
            pl.BlockSpec((1, d_z), lambda i: (0, 0)),
            full(Wk),
            pl.BlockSpec((1, d_z), lambda i: (0, 0)),
            full(Ww),
            pl.BlockSpec((1, 1), lambda i: (0, 0)),
        ],
        out_specs=[
            pl.BlockSpec((blk, d_z), lambda i: (i, 0)),
            pl.BlockSpec((blk, d_z), lambda i: (i, 0)),
            pl.BlockSpec((blk, 1), lambda i: (i, 0)),
        ],
        out_shape=[
            jax.ShapeDtypeStruct((n, d_z), jnp.float32),
            jax.ShapeDtypeStruct((n, d_z), jnp.float32),
            jax.ShapeDtypeStruct((n, 1), jnp.float32),
        ],
    )(x, s_part, deg_part, W_self, W_nbr, b_gnn.reshape(1, d), W1,
      b1.reshape(1, d), W2, b2.reshape(1, d_z), Wk, bk.reshape(1, d_z), Ww,
      bw.reshape(1, 1))
    return (z, keys, wout)
